# baseline (speedup 1.0000x reference)
import functools

import jax
import jax.numpy as jnp
import numpy as np
from jax import lax
from jax.experimental import pallas as pl
from jax.experimental.pallas import tpu as pltpu
from jax.experimental.pallas import tpu_sc as plsc

N = 10000
E = 320000
D = 128
HID = 64
N_PAD = 10240
NC = 2
NS = 16
CH = 128
CE = 128
NSUB = CE // CH
EPW = 10240
E_PAD = NC * NS * EPW
NCHUNK = EPW // CE
ROWS_PT = N_PAD // NS
RB = 1024


def _proj_call(K):
    def body(x_ref, w_ref, a_ref, b_ref, fe_ref, er_ref):
        feat = jnp.dot(x_ref[...], w_ref[...], preferred_element_type=jnp.float32)
        el = jnp.dot(feat, a_ref[...], preferred_element_type=jnp.float32)
        er = jnp.dot(feat, b_ref[...], preferred_element_type=jnp.float32)
        fe_ref[...] = jnp.concatenate([feat, el, el], axis=1)
        er_ref[...] = er

    return pl.pallas_call(
        body,
        grid=(N_PAD // RB,),
        in_specs=[
            pl.BlockSpec((RB, K), lambda i: (i, 0)),
            pl.BlockSpec((K, HID), lambda i: (0, 0)),
            pl.BlockSpec((HID, 8), lambda i: (0, 0)),
            pl.BlockSpec((HID, 16), lambda i: (0, 0)),
        ],
        out_specs=[
            pl.BlockSpec((RB, 80), lambda i: (i, 0)),
            pl.BlockSpec((RB, 16), lambda i: (i, 0)),
        ],
        out_shape=[
            jax.ShapeDtypeStruct((N_PAD, 80), jnp.float32),
            jax.ShapeDtypeStruct((N_PAD, 16), jnp.float32),
        ],
    )


def _norm_call(apply_elu):
    def body(num_ref, den_ref, b_ref, e_ref, o_ref):
        nm = num_ref[0] + num_ref[1]
        dn = den_ref[0] + den_ref[1]
        den64 = jnp.dot(dn, e_ref[...], preferred_element_type=jnp.float32)
        v = nm / (den64 + 1e-9) + b_ref[...]
        if apply_elu:
            v = jnp.where(v > 0, v, jnp.exp(v) - 1.0)
        o_ref[...] = v

    return pl.pallas_call(
        body,
        grid=(N_PAD // RB,),
        in_specs=[
            pl.BlockSpec((2, RB, HID), lambda i: (0, i, 0)),
            pl.BlockSpec((2, RB, 16), lambda i: (0, i, 0)),
            pl.BlockSpec((1, HID), lambda i: (0, 0)),
            pl.BlockSpec((16, HID), lambda i: (0, 0)),
        ],
        out_specs=pl.BlockSpec((RB, HID), lambda i: (i, 0)),
        out_shape=jax.ShapeDtypeStruct((N_PAD, HID), jnp.float32),
    )


_EXP_A = False
_EXP_B = False
_EXP_C = False
_EXP_D = True

def _edge_call(mode):
    mesh = plsc.VectorSubcoreMesh(core_axis_name="c", subcore_axis_name="s")

    @functools.partial(
        pl.kernel,
        out_type=(
            jax.ShapeDtypeStruct((NC, N_PAD, 64), jnp.float32),
            jax.ShapeDtypeStruct((NC, N_PAD, 16), jnp.float32),
        ),
        mesh=mesh,
        compiler_params=pltpu.CompilerParams(
            needs_layout_passes=False, use_tc_tiling_on_sc=False),
        scratch_types=[
            pltpu.VMEM((NSUB, CH), jnp.int32),
            pltpu.VMEM((NSUB, CH), jnp.int32),
            pltpu.VMEM((CE, 80), jnp.float32),
            pltpu.VMEM((CE, 16), jnp.float32),
            pltpu.VMEM((CE, 16), jnp.float32),
            pltpu.VMEM((CE, 64), jnp.float32),
            pltpu.VMEM((NSUB, CH), jnp.int32),
            pltpu.VMEM((NSUB, CH), jnp.int32),
            pltpu.VMEM((CE, 80), jnp.float32),
            pltpu.VMEM((CE, 16), jnp.float32),
            pltpu.VMEM((CE, 16), jnp.float32),
            pltpu.VMEM((CE, 64), jnp.float32),
            pltpu.VMEM_SHARED((N_PAD, 64), jnp.float32),
            pltpu.VMEM_SHARED((N_PAD, 16), jnp.float32),
            pltpu.SemaphoreType.DMA,
            pltpu.SemaphoreType.DMA,
            pltpu.SemaphoreType.DMA,
            pltpu.SemaphoreType.DMA,
        ],
    )
    def k(src_h, dst_h, fe_h, er_h, num_o, den_o,
          src_v0, dst_v0, fe_v0, er_v0, w_v0, msg_v0,
          src_v1, dst_v1, fe_v1, er_v1, w_v1, msg_v1,
          num_sp, den_sp, gsem0, gsem1, ssem0, ssem1):
        c = lax.axis_index("c")
        s = lax.axis_index("s")
        wid = s * NC + c
        ebase = wid * EPW
        zero16 = jnp.zeros((16,), jnp.float32)
        iota = lax.iota(jnp.int32, 16)
        if mode == 1:
            patt = [(q * 16 + iota) >> 3 for q in range(4)]
        else:
            patt = [jnp.zeros((16,), jnp.int32) for _ in range(4)]
        bufs = [(src_v0, dst_v0, fe_v0, er_v0, w_v0, msg_v0, gsem0, ssem0),
                (src_v1, dst_v1, fe_v1, er_v1, w_v1, msg_v1, gsem1, ssem1)]

        @plsc.parallel_loop(0, CE * 4, unroll=8)
        def znloop(i):
            msg_v0[i >> 2, pl.ds((i & 3) * 16, 16)] = zero16

        @plsc.parallel_loop(0, CE, unroll=8)
        def zdloop(i):
            w_v0[i, :] = zero16

        row0 = s * ROWS_PT
        done = 0
        while done < ROWS_PT:
            step = min(CE, ROWS_PT - done)
            pltpu.sync_copy(msg_v0.at[pl.ds(0, step)],
                            num_sp.at[pl.ds(row0 + done, step)])
            pltpu.sync_copy(w_v0.at[pl.ds(0, step)],
                            den_sp.at[pl.ds(row0 + done, step)])
            done += step
        plsc.subcore_barrier()

        def load_idx(g, b):
            src_v, dst_v = bufs[b][0], bufs[b][1]
            for j in range(NSUB):
                base = ebase + g * CE + j * CH
                pltpu.sync_copy(src_h.at[pl.ds(base, CH)], src_v.at[j])
                pltpu.sync_copy(dst_h.at[pl.ds(base, CH)], dst_v.at[j])

        def gathers(b):
            src_v, dst_v, fe_v, er_v = bufs[b][0], bufs[b][1], bufs[b][2], bufs[b][3]
            gsem = bufs[b][6]
            out = []
            for j in range(NSUB):
                if _EXP_D:
                    for hh in range(2):
                        out.append(pltpu.make_async_copy(
                            fe_h.at[src_v.at[j, pl.ds(hh * 64, 64)]],
                            fe_v.at[pl.ds(j * CH + hh * 64, 64)], gsem))
                else:
                    out.append(pltpu.make_async_copy(
                        fe_h.at[src_v.at[j]], fe_v.at[pl.ds(j * CH, CH)], gsem))
                out.append(pltpu.make_async_copy(
                    er_h.at[dst_v.at[j]], er_v.at[pl.ds(j * CH, CH)], gsem))
            return out

        def scatters(b):
            dst_v, w_v, msg_v = bufs[b][1], bufs[b][4], bufs[b][5]
            ssem = bufs[b][7]
            out = []
            for j in range(NSUB):
                out.append(pltpu.make_async_copy(
                    msg_v.at[pl.ds(j * CH, CH)], num_sp.at[dst_v.at[j]], ssem))
                out.append(pltpu.make_async_copy(
                    w_v.at[pl.ds(j * CH, CH)], den_sp.at[dst_v.at[j]], ssem))
            return out

        def compute(b):
            fe_v, er_v, w_v, msg_v = bufs[b][2], bufs[b][3], bufs[b][4], bufs[b][5]

            @plsc.parallel_loop(0, CE, unroll=4)
            def rows(r):
                z = fe_v[r, pl.ds(64, 16)] + er_v[r, :]
                w = jnp.exp(jnp.where(z > 0, z, 0.2 * z))
                w_v[r, :] = w
                for q in range(4):
                    wq = jnp.take_along_axis(w, patt[q], axis=0)
                    msg_v[r, pl.ds(q * 16, 16)] = fe_v[r, pl.ds(q * 16, 16)] * wq

        load_idx(0, 0)
        if not _EXP_C:
            for cp in gathers(0):
                cp.start()

        def halfstep(i, b):
            g = 2 * i + b
            nb2 = 1 - b

            @pl.when(g >= 1)
            def _():
                if not _EXP_A:
                    for cp in scatters(nb2):
                        cp.wait()

            @pl.when(g + 1 < NCHUNK)
            def _():
                load_idx(g + 1, nb2)
                if not _EXP_C:
                    for cp in gathers(nb2):
                        cp.start()

            if not _EXP_C:
                for cp in gathers(b):
                    cp.wait()
            if not _EXP_B:
                compute(b)
            if not _EXP_A:
                for cp in scatters(b):
                    cp.start(add=True)

        def pipe(i, _):
            halfstep(i, 0)
            halfstep(i, 1)
            return 0
        lax.fori_loop(0, NCHUNK // 2, pipe, 0)

        if not _EXP_A:
            for cp in scatters((NCHUNK - 1) & 1):
                cp.wait()
        plsc.subcore_barrier()

        pltpu.sync_copy(num_sp.at[pl.ds(row0, ROWS_PT)],
                        num_o.at[c, pl.ds(row0, ROWS_PT)])
        pltpu.sync_copy(den_sp.at[pl.ds(row0, ROWS_PT)],
                        den_o.at[c, pl.ds(row0, ROWS_PT)])

    return k


_E16_L1 = np.zeros((16, HID), np.float32)
for _h in range(8):
    _E16_L1[_h, _h * 8:(_h + 1) * 8] = 1.0
_E16_L2 = np.zeros((16, HID), np.float32)
_E16_L2[0, :] = 1.0


def _blockdiag(a):
    return (jnp.eye(8, dtype=a.dtype)[:, None, :] * a[:, :, None]).reshape(HID, 8)


def kernel(x, edge_index, W1, al1, ar1, b1, W2, al2, ar2, b2):
    src = jnp.concatenate(
        [edge_index[0].astype(jnp.int32), jnp.full((E_PAD - E,), N, jnp.int32)])
    dst = jnp.concatenate(
        [edge_index[1].astype(jnp.int32), jnp.full((E_PAD - E,), N, jnp.int32)])
    xp = jnp.zeros((N_PAD, D), jnp.float32).at[:N].set(x)

    A1 = _blockdiag(al1)
    B1 = jnp.concatenate([_blockdiag(ar1)] * 2, axis=1)
    A2 = jnp.tile(al2.T, (1, 8))
    B2 = jnp.tile(ar2.T, (1, 16))
    E1 = jnp.asarray(_E16_L1)
    E2 = jnp.asarray(_E16_L2)

    fe1, er1 = _proj_call(D)(xp, W1, A1, B1)
    num1, den1 = _edge_call(1)(src, dst, fe1, er1)
    h1 = _norm_call(True)(num1, den1, b1.reshape(1, HID), E1)
    fe2, er2 = _proj_call(HID)(h1, W2, A2, B2)
    num2, den2 = _edge_call(2)(src, dst, fe2, er2)
    out = _norm_call(False)(num2, den2, b2.reshape(1, HID), E2)
    return out[:N]


# speedup vs baseline: 1.7743x; 1.7743x over previous
import functools

import jax
import jax.numpy as jnp
import numpy as np
from jax import lax
from jax.experimental import pallas as pl
from jax.experimental.pallas import tpu as pltpu
from jax.experimental.pallas import tpu_sc as plsc

N = 10000
E = 320000
D = 128
HID = 64
N_PAD = 10240
NC = 2
NS = 16
CH = 128
CE = 128
NSUB = CE // CH
EPW = 10240
E_PAD = NC * NS * EPW
NCHUNK = EPW // CE
ROWS_PT = N_PAD // NS
RB = 1024


def _proj_call(K):
    def body(x_ref, w_ref, a_ref, b_ref, ei_ref, fb_ref, er_ref):
        feat = jnp.dot(x_ref[...], w_ref[...], preferred_element_type=jnp.float32)
        el = jnp.dot(feat, a_ref[...], preferred_element_type=jnp.float32)
        er = jnp.dot(feat, b_ref[...], preferred_element_type=jnp.float32)
        eli = jnp.dot(el, ei_ref[...], preferred_element_type=jnp.float32)
        fb_ref[...] = jnp.concatenate([feat, eli], axis=1).astype(jnp.bfloat16)
        er_ref[...] = er

    return pl.pallas_call(
        body,
        grid=(N_PAD // RB,),
        in_specs=[
            pl.BlockSpec((RB, K), lambda i: (i, 0)),
            pl.BlockSpec((K, HID), lambda i: (0, 0)),
            pl.BlockSpec((HID, 8), lambda i: (0, 0)),
            pl.BlockSpec((HID, 16), lambda i: (0, 0)),
            pl.BlockSpec((8, 16), lambda i: (0, 0)),
        ],
        out_specs=[
            pl.BlockSpec((RB, 80), lambda i: (i, 0)),
            pl.BlockSpec((RB, 16), lambda i: (i, 0)),
        ],
        out_shape=[
            jax.ShapeDtypeStruct((N_PAD, 80), jnp.bfloat16),
            jax.ShapeDtypeStruct((N_PAD, 16), jnp.float32),
        ],
    )


def _norm_call(apply_elu):
    def body(num_ref, den_ref, b_ref, e_ref, p_ref, o_ref):
        nm = num_ref[0] + num_ref[1]
        dn = den_ref[0] + den_ref[1]
        nmp = jnp.dot(nm, p_ref[...], preferred_element_type=jnp.float32)
        den64 = jnp.dot(dn, e_ref[...], preferred_element_type=jnp.float32)
        v = nmp / (den64 + 1e-9) + b_ref[...]
        if apply_elu:
            v = jnp.where(v > 0, v, jnp.exp(v) - 1.0)
        o_ref[...] = v

    return pl.pallas_call(
        body,
        grid=(N_PAD // RB,),
        in_specs=[
            pl.BlockSpec((2, RB, HID), lambda i: (0, i, 0)),
            pl.BlockSpec((2, RB, 16), lambda i: (0, i, 0)),
            pl.BlockSpec((1, HID), lambda i: (0, 0)),
            pl.BlockSpec((16, HID), lambda i: (0, 0)),
            pl.BlockSpec((HID, HID), lambda i: (0, 0)),
        ],
        out_specs=pl.BlockSpec((RB, HID), lambda i: (i, 0)),
        out_shape=jax.ShapeDtypeStruct((N_PAD, HID), jnp.float32),
    )


_EXP_A = False
_EXP_B = False
def _edge_call(mode):
    mesh = plsc.VectorSubcoreMesh(core_axis_name="c", subcore_axis_name="s")

    @functools.partial(
        pl.kernel,
        out_type=(
            jax.ShapeDtypeStruct((NC, N_PAD, 64), jnp.float32),
            jax.ShapeDtypeStruct((NC, N_PAD, 16), jnp.float32),
        ),
        mesh=mesh,
        compiler_params=pltpu.CompilerParams(
            needs_layout_passes=False, use_tc_tiling_on_sc=False),
        scratch_types=[
            pltpu.VMEM((NSUB, CH), jnp.int32),
            pltpu.VMEM((NSUB, CH), jnp.int32),
            pltpu.VMEM((CE, 80), jnp.bfloat16),
            pltpu.VMEM((CE, 16), jnp.float32),
            pltpu.VMEM((CE, 16), jnp.float32),
            pltpu.VMEM((CE, 64), jnp.float32),
            pltpu.VMEM((NSUB, CH), jnp.int32),
            pltpu.VMEM((NSUB, CH), jnp.int32),
            pltpu.VMEM((CE, 80), jnp.bfloat16),
            pltpu.VMEM((CE, 16), jnp.float32),
            pltpu.VMEM((CE, 16), jnp.float32),
            pltpu.VMEM((CE, 64), jnp.float32),
            pltpu.VMEM_SHARED((N_PAD, 64), jnp.float32),
            pltpu.VMEM_SHARED((N_PAD, 16), jnp.float32),
            pltpu.SemaphoreType.DMA,
            pltpu.SemaphoreType.DMA,
            pltpu.SemaphoreType.DMA,
            pltpu.SemaphoreType.DMA,
        ],
    )
    def k(src_h, dst_h, fe_h, er_h, num_o, den_o,
          src_v0, dst_v0, fe_v0, er_v0, w_v0, msg_v0,
          src_v1, dst_v1, fe_v1, er_v1, w_v1, msg_v1,
          num_sp, den_sp, gsem0, gsem1, ssem0, ssem1):
        c = lax.axis_index("c")
        s = lax.axis_index("s")
        wid = s * NC + c
        ebase = wid * EPW
        zero16 = jnp.zeros((16,), jnp.float32)
        iota = lax.iota(jnp.int32, 16)
        if mode == 1:
            patt = [8 + 4 * q + (iota >> 2) for q in range(2)]
        else:
            patt = [8 + jnp.zeros((16,), jnp.int32) for _ in range(2)]
        bufs = [(src_v0, dst_v0, fe_v0, er_v0, w_v0, msg_v0, gsem0, ssem0),
                (src_v1, dst_v1, fe_v1, er_v1, w_v1, msg_v1, gsem1, ssem1)]

        @plsc.parallel_loop(0, CE * 4, unroll=8)
        def znloop(i):
            msg_v0[i >> 2, pl.ds((i & 3) * 16, 16)] = zero16

        @plsc.parallel_loop(0, CE, unroll=8)
        def zdloop(i):
            w_v0[i, :] = zero16

        row0 = s * ROWS_PT
        done = 0
        while done < ROWS_PT:
            step = min(CE, ROWS_PT - done)
            pltpu.sync_copy(msg_v0.at[pl.ds(0, step)],
                            num_sp.at[pl.ds(row0 + done, step)])
            pltpu.sync_copy(w_v0.at[pl.ds(0, step)],
                            den_sp.at[pl.ds(row0 + done, step)])
            done += step
        plsc.subcore_barrier()

        def load_idx(g, b):
            src_v, dst_v = bufs[b][0], bufs[b][1]
            for j in range(NSUB):
                base = ebase + g * CE + j * CH
                pltpu.sync_copy(src_h.at[pl.ds(base, CH)], src_v.at[j])
                pltpu.sync_copy(dst_h.at[pl.ds(base, CH)], dst_v.at[j])

        def gathers(b):
            src_v, dst_v, fe_v, er_v = bufs[b][0], bufs[b][1], bufs[b][2], bufs[b][3]
            gsem = bufs[b][6]
            out = []
            for j in range(NSUB):
                out.append(pltpu.make_async_copy(
                    fe_h.at[src_v.at[j]], fe_v.at[pl.ds(j * CH, CH)], gsem))
                out.append(pltpu.make_async_copy(
                    er_h.at[dst_v.at[j]], er_v.at[pl.ds(j * CH, CH)], gsem))
            return out

        def scatters(b):
            dst_v, w_v, msg_v = bufs[b][1], bufs[b][4], bufs[b][5]
            ssem = bufs[b][7]
            out = []
            for j in range(NSUB):
                out.append(pltpu.make_async_copy(
                    msg_v.at[pl.ds(j * CH, CH)], num_sp.at[dst_v.at[j]], ssem))
                out.append(pltpu.make_async_copy(
                    w_v.at[pl.ds(j * CH, CH)], den_sp.at[dst_v.at[j]], ssem))
            return out

        def compute(b):
            fe_v, er_v, w_v, msg_v = bufs[b][2], bufs[b][3], bufs[b][4], bufs[b][5]

            @plsc.parallel_loop(0, CE, unroll=4)
            def rows(r):
                ue = fe_v[r, pl.ds(48, 32)]
                ea, _eb = plsc.unpack(ue, format=plsc.PackFormat.INTERLEAVED,
                                      preferred_element_type=jnp.float32)
                z = ea + er_v[r, :]
                w = jnp.exp(jnp.where(z > 0, z, 0.2 * z))
                w_v[r, :] = w
                for q in range(2):
                    uf = fe_v[r, pl.ds(q * 32, 32)]
                    fa, fb2 = plsc.unpack(uf, format=plsc.PackFormat.INTERLEAVED,
                                          preferred_element_type=jnp.float32)
                    wq = jnp.take_along_axis(w, patt[q], axis=0)
                    msg_v[r, pl.ds(q * 32, 16)] = fa * wq
                    msg_v[r, pl.ds(q * 32 + 16, 16)] = fb2 * wq

        load_idx(0, 0)
        for cp in gathers(0):
            cp.start()

        def halfstep(i, b):
            g = 2 * i + b
            nb2 = 1 - b

            @pl.when(g >= 1)
            def _():
                for cp in scatters(nb2):
                    cp.wait()

            @pl.when(g + 1 < NCHUNK)
            def _():
                load_idx(g + 1, nb2)
                for cp in gathers(nb2):
                    cp.start()

            for cp in gathers(b):
                cp.wait()
            compute(b)
            for cp in scatters(b):
                cp.start(add=True)

        def pipe(i, _):
            halfstep(i, 0)
            halfstep(i, 1)
            return 0
        lax.fori_loop(0, NCHUNK // 2, pipe, 0)

        for cp in scatters((NCHUNK - 1) & 1):
            cp.wait()
        plsc.subcore_barrier()

        pltpu.sync_copy(num_sp.at[pl.ds(row0, ROWS_PT)],
                        num_o.at[c, pl.ds(row0, ROWS_PT)])
        pltpu.sync_copy(den_sp.at[pl.ds(row0, ROWS_PT)],
                        den_o.at[c, pl.ds(row0, ROWS_PT)])

    return k


_E16_L1 = np.zeros((16, HID), np.float32)
for _h in range(8):
    _E16_L1[8 + _h, _h * 8:(_h + 1) * 8] = 1.0
_E16_L2 = np.zeros((16, HID), np.float32)
_E16_L2[8, :] = 1.0

_PERM = np.zeros((HID, HID), np.float32)
for _c in range(HID):
    _q, _rc = _c // 32, _c % 32
    _f = 32 * _q + (2 * _rc if _rc < 16 else 2 * (_rc - 16) + 1)
    _PERM[_c, _f] = 1.0

_EI = np.zeros((8, 16), np.float32)
for _j in range(8):
    _EI[_j, 2 * _j] = 1.0
    _EI[_j, 2 * _j + 1] = 1.0


def _blockdiag(a):
    return (jnp.eye(8, dtype=a.dtype)[:, None, :] * a[:, :, None]).reshape(HID, 8)


def kernel(x, edge_index, W1, al1, ar1, b1, W2, al2, ar2, b2):
    src = jnp.concatenate(
        [edge_index[0].astype(jnp.int32), jnp.full((E_PAD - E,), N, jnp.int32)])
    dst = jnp.concatenate(
        [edge_index[1].astype(jnp.int32), jnp.full((E_PAD - E,), N, jnp.int32)])
    xp = jnp.zeros((N_PAD, D), jnp.float32).at[:N].set(x)

    A1 = _blockdiag(al1)
    B1 = jnp.concatenate([_blockdiag(ar1)] * 2, axis=1)
    A2 = jnp.tile(al2.T, (1, 8))
    B2 = jnp.tile(ar2.T, (1, 16))
    E1 = jnp.asarray(_E16_L1)
    E2 = jnp.asarray(_E16_L2)
    P = jnp.asarray(_PERM)
    EI = jnp.asarray(_EI)

    fb1, er1 = _proj_call(D)(xp, W1, A1, B1, EI)
    num1, den1 = _edge_call(1)(src, dst, fb1, er1)
    h1 = _norm_call(True)(num1, den1, b1.reshape(1, HID), E1, P)
    fb2, er2 = _proj_call(HID)(h1, W2, A2, B2, EI)
    num2, den2 = _edge_call(2)(src, dst, fb2, er2)
    out = _norm_call(False)(num2, den2, b2.reshape(1, HID), E2, P)
    return out[:N]


# speedup vs baseline: 1.7759x; 1.0009x over previous
import functools

import jax
import jax.numpy as jnp
import numpy as np
from jax import lax
from jax.experimental import pallas as pl
from jax.experimental.pallas import tpu as pltpu
from jax.experimental.pallas import tpu_sc as plsc

N = 10000
E = 320000
D = 128
HID = 64
N_PAD = 10240
NC = 2
NS = 16
CH = 128
CE = 256
NSUB = CE // CH
EPW = 10240
E_PAD = NC * NS * EPW
NCHUNK = EPW // CE
ROWS_PT = N_PAD // NS
RB = 1024


def _proj_call(K):
    def body(x_ref, w_ref, a_ref, b_ref, ei_ref, fb_ref, er_ref):
        feat = jnp.dot(x_ref[...], w_ref[...], preferred_element_type=jnp.float32)
        el = jnp.dot(feat, a_ref[...], preferred_element_type=jnp.float32)
        er = jnp.dot(feat, b_ref[...], preferred_element_type=jnp.float32)
        eli = jnp.dot(el, ei_ref[...], preferred_element_type=jnp.float32)
        fb_ref[...] = jnp.concatenate([feat, eli], axis=1).astype(jnp.bfloat16)
        er_ref[...] = er

    return pl.pallas_call(
        body,
        grid=(N_PAD // RB,),
        in_specs=[
            pl.BlockSpec((RB, K), lambda i: (i, 0)),
            pl.BlockSpec((K, HID), lambda i: (0, 0)),
            pl.BlockSpec((HID, 8), lambda i: (0, 0)),
            pl.BlockSpec((HID, 16), lambda i: (0, 0)),
            pl.BlockSpec((8, 16), lambda i: (0, 0)),
        ],
        out_specs=[
            pl.BlockSpec((RB, 80), lambda i: (i, 0)),
            pl.BlockSpec((RB, 16), lambda i: (i, 0)),
        ],
        out_shape=[
            jax.ShapeDtypeStruct((N_PAD, 80), jnp.bfloat16),
            jax.ShapeDtypeStruct((N_PAD, 16), jnp.float32),
        ],
    )


def _norm_call(apply_elu):
    def body(num_ref, den_ref, b_ref, e_ref, p_ref, o_ref):
        nm = num_ref[0] + num_ref[1]
        dn = den_ref[0] + den_ref[1]
        nmp = jnp.dot(nm, p_ref[...], preferred_element_type=jnp.float32)
        den64 = jnp.dot(dn, e_ref[...], preferred_element_type=jnp.float32)
        v = nmp / (den64 + 1e-9) + b_ref[...]
        if apply_elu:
            v = jnp.where(v > 0, v, jnp.exp(v) - 1.0)
        o_ref[...] = v

    return pl.pallas_call(
        body,
        grid=(N_PAD // RB,),
        in_specs=[
            pl.BlockSpec((2, RB, HID), lambda i: (0, i, 0)),
            pl.BlockSpec((2, RB, 16), lambda i: (0, i, 0)),
            pl.BlockSpec((1, HID), lambda i: (0, 0)),
            pl.BlockSpec((16, HID), lambda i: (0, 0)),
            pl.BlockSpec((HID, HID), lambda i: (0, 0)),
        ],
        out_specs=pl.BlockSpec((RB, HID), lambda i: (i, 0)),
        out_shape=jax.ShapeDtypeStruct((N_PAD, HID), jnp.float32),
    )


_EXP_A = False
_EXP_B = False
def _edge_call(mode):
    mesh = plsc.VectorSubcoreMesh(core_axis_name="c", subcore_axis_name="s")

    @functools.partial(
        pl.kernel,
        out_type=(
            jax.ShapeDtypeStruct((NC, N_PAD, 64), jnp.float32),
            jax.ShapeDtypeStruct((NC, N_PAD, 16), jnp.float32),
        ),
        mesh=mesh,
        compiler_params=pltpu.CompilerParams(
            needs_layout_passes=False, use_tc_tiling_on_sc=False),
        scratch_types=[
            pltpu.VMEM((NSUB, CH), jnp.int32),
            pltpu.VMEM((NSUB, CH), jnp.int32),
            pltpu.VMEM((CE, 80), jnp.bfloat16),
            pltpu.VMEM((CE, 16), jnp.float32),
            pltpu.VMEM((CE, 16), jnp.float32),
            pltpu.VMEM((CE, 64), jnp.float32),
            pltpu.VMEM((NSUB, CH), jnp.int32),
            pltpu.VMEM((NSUB, CH), jnp.int32),
            pltpu.VMEM((CE, 80), jnp.bfloat16),
            pltpu.VMEM((CE, 16), jnp.float32),
            pltpu.VMEM((CE, 16), jnp.float32),
            pltpu.VMEM((CE, 64), jnp.float32),
            pltpu.VMEM_SHARED((N_PAD, 64), jnp.float32),
            pltpu.VMEM_SHARED((N_PAD, 16), jnp.float32),
            pltpu.SemaphoreType.DMA,
            pltpu.SemaphoreType.DMA,
            pltpu.SemaphoreType.DMA,
            pltpu.SemaphoreType.DMA,
        ],
    )
    def k(src_h, dst_h, fe_h, er_h, num_o, den_o,
          src_v0, dst_v0, fe_v0, er_v0, w_v0, msg_v0,
          src_v1, dst_v1, fe_v1, er_v1, w_v1, msg_v1,
          num_sp, den_sp, gsem0, gsem1, ssem0, ssem1):
        c = lax.axis_index("c")
        s = lax.axis_index("s")
        wid = s * NC + c
        ebase = wid * EPW
        zero16 = jnp.zeros((16,), jnp.float32)
        iota = lax.iota(jnp.int32, 16)
        if mode == 1:
            patt = [8 + 4 * q + (iota >> 2) for q in range(2)]
        else:
            patt = [8 + jnp.zeros((16,), jnp.int32) for _ in range(2)]
        bufs = [(src_v0, dst_v0, fe_v0, er_v0, w_v0, msg_v0, gsem0, ssem0),
                (src_v1, dst_v1, fe_v1, er_v1, w_v1, msg_v1, gsem1, ssem1)]

        @plsc.parallel_loop(0, CE * 4, unroll=8)
        def znloop(i):
            msg_v0[i >> 2, pl.ds((i & 3) * 16, 16)] = zero16

        @plsc.parallel_loop(0, CE, unroll=8)
        def zdloop(i):
            w_v0[i, :] = zero16

        row0 = s * ROWS_PT
        done = 0
        while done < ROWS_PT:
            step = min(CE, ROWS_PT - done)
            pltpu.sync_copy(msg_v0.at[pl.ds(0, step)],
                            num_sp.at[pl.ds(row0 + done, step)])
            pltpu.sync_copy(w_v0.at[pl.ds(0, step)],
                            den_sp.at[pl.ds(row0 + done, step)])
            done += step
        plsc.subcore_barrier()

        def load_idx(g, b):
            src_v, dst_v = bufs[b][0], bufs[b][1]
            for j in range(NSUB):
                base = ebase + g * CE + j * CH
                pltpu.sync_copy(src_h.at[pl.ds(base, CH)], src_v.at[j])
                pltpu.sync_copy(dst_h.at[pl.ds(base, CH)], dst_v.at[j])

        def gathers(b):
            src_v, dst_v, fe_v, er_v = bufs[b][0], bufs[b][1], bufs[b][2], bufs[b][3]
            gsem = bufs[b][6]
            out = []
            for j in range(NSUB):
                out.append(pltpu.make_async_copy(
                    fe_h.at[src_v.at[j]], fe_v.at[pl.ds(j * CH, CH)], gsem))
                out.append(pltpu.make_async_copy(
                    er_h.at[dst_v.at[j]], er_v.at[pl.ds(j * CH, CH)], gsem))
            return out

        def scatters(b):
            dst_v, w_v, msg_v = bufs[b][1], bufs[b][4], bufs[b][5]
            ssem = bufs[b][7]
            out = []
            for j in range(NSUB):
                out.append(pltpu.make_async_copy(
                    msg_v.at[pl.ds(j * CH, CH)], num_sp.at[dst_v.at[j]], ssem))
                out.append(pltpu.make_async_copy(
                    w_v.at[pl.ds(j * CH, CH)], den_sp.at[dst_v.at[j]], ssem))
            return out

        def compute(b):
            fe_v, er_v, w_v, msg_v = bufs[b][2], bufs[b][3], bufs[b][4], bufs[b][5]

            @plsc.parallel_loop(0, CE, unroll=4)
            def rows(r):
                ue = fe_v[r, pl.ds(48, 32)]
                ea, _eb = plsc.unpack(ue, format=plsc.PackFormat.INTERLEAVED,
                                      preferred_element_type=jnp.float32)
                z = ea + er_v[r, :]
                w = jnp.exp(jnp.where(z > 0, z, 0.2 * z))
                w_v[r, :] = w
                for q in range(2):
                    uf = fe_v[r, pl.ds(q * 32, 32)]
                    fa, fb2 = plsc.unpack(uf, format=plsc.PackFormat.INTERLEAVED,
                                          preferred_element_type=jnp.float32)
                    wq = jnp.take_along_axis(w, patt[q], axis=0)
                    msg_v[r, pl.ds(q * 32, 16)] = fa * wq
                    msg_v[r, pl.ds(q * 32 + 16, 16)] = fb2 * wq

        load_idx(0, 0)
        for cp in gathers(0):
            cp.start()

        def halfstep(i, b):
            g = 2 * i + b
            nb2 = 1 - b

            @pl.when(g >= 1)
            def _():
                for cp in scatters(nb2):
                    cp.wait()

            @pl.when(g + 1 < NCHUNK)
            def _():
                load_idx(g + 1, nb2)
                for cp in gathers(nb2):
                    cp.start()

            for cp in gathers(b):
                cp.wait()
            compute(b)
            for cp in scatters(b):
                cp.start(add=True)

        def pipe(i, _):
            halfstep(i, 0)
            halfstep(i, 1)
            return 0
        lax.fori_loop(0, NCHUNK // 2, pipe, 0)

        for cp in scatters((NCHUNK - 1) & 1):
            cp.wait()
        plsc.subcore_barrier()

        pltpu.sync_copy(num_sp.at[pl.ds(row0, ROWS_PT)],
                        num_o.at[c, pl.ds(row0, ROWS_PT)])
        pltpu.sync_copy(den_sp.at[pl.ds(row0, ROWS_PT)],
                        den_o.at[c, pl.ds(row0, ROWS_PT)])

    return k


_E16_L1 = np.zeros((16, HID), np.float32)
for _h in range(8):
    _E16_L1[8 + _h, _h * 8:(_h + 1) * 8] = 1.0
_E16_L2 = np.zeros((16, HID), np.float32)
_E16_L2[8, :] = 1.0

_PERM = np.zeros((HID, HID), np.float32)
for _c in range(HID):
    _q, _rc = _c // 32, _c % 32
    _f = 32 * _q + (2 * _rc if _rc < 16 else 2 * (_rc - 16) + 1)
    _PERM[_c, _f] = 1.0

_EI = np.zeros((8, 16), np.float32)
for _j in range(8):
    _EI[_j, 2 * _j] = 1.0
    _EI[_j, 2 * _j + 1] = 1.0


def _blockdiag(a):
    return (jnp.eye(8, dtype=a.dtype)[:, None, :] * a[:, :, None]).reshape(HID, 8)


def kernel(x, edge_index, W1, al1, ar1, b1, W2, al2, ar2, b2):
    src = jnp.concatenate(
        [edge_index[0].astype(jnp.int32), jnp.full((E_PAD - E,), N, jnp.int32)])
    dst = jnp.concatenate(
        [edge_index[1].astype(jnp.int32), jnp.full((E_PAD - E,), N, jnp.int32)])
    xp = jnp.zeros((N_PAD, D), jnp.float32).at[:N].set(x)

    A1 = _blockdiag(al1)
    B1 = jnp.concatenate([_blockdiag(ar1)] * 2, axis=1)
    A2 = jnp.tile(al2.T, (1, 8))
    B2 = jnp.tile(ar2.T, (1, 16))
    E1 = jnp.asarray(_E16_L1)
    E2 = jnp.asarray(_E16_L2)
    P = jnp.asarray(_PERM)
    EI = jnp.asarray(_EI)

    fb1, er1 = _proj_call(D)(xp, W1, A1, B1, EI)
    num1, den1 = _edge_call(1)(src, dst, fb1, er1)
    h1 = _norm_call(True)(num1, den1, b1.reshape(1, HID), E1, P)
    fb2, er2 = _proj_call(HID)(h1, W2, A2, B2, EI)
    num2, den2 = _edge_call(2)(src, dst, fb2, er2)
    out = _norm_call(False)(num2, den2, b2.reshape(1, HID), E2, P)
    return out[:N]


# speedup vs baseline: 1.7965x; 1.0116x over previous
import functools

import jax
import jax.numpy as jnp
import numpy as np
from jax import lax
from jax.experimental import pallas as pl
from jax.experimental.pallas import tpu as pltpu
from jax.experimental.pallas import tpu_sc as plsc

N = 10000
E = 320000
D = 128
HID = 64
N_PAD = 10240
NC = 2
NS = 16
CH = 128
CE = 256
NSUB = CE // CH
EPW = 10240
E_PAD = NC * NS * EPW
NCHUNK = EPW // CE
ROWS_PT = N_PAD // NS
RB = 1024


def _proj_call(K):
    def body(x_ref, w_ref, a_ref, b_ref, ei_ref, fb_ref, er_ref):
        feat = jnp.dot(x_ref[...], w_ref[...], preferred_element_type=jnp.float32)
        el = jnp.dot(feat, a_ref[...], preferred_element_type=jnp.float32)
        er = jnp.dot(feat, b_ref[...], preferred_element_type=jnp.float32)
        eli = jnp.dot(el, ei_ref[...], preferred_element_type=jnp.float32)
        fb_ref[...] = jnp.concatenate([feat, eli], axis=1).astype(jnp.bfloat16)
        er_ref[...] = er

    return pl.pallas_call(
        body,
        grid=(N_PAD // RB,),
        in_specs=[
            pl.BlockSpec((RB, K), lambda i: (i, 0)),
            pl.BlockSpec((K, HID), lambda i: (0, 0)),
            pl.BlockSpec((HID, 8), lambda i: (0, 0)),
            pl.BlockSpec((HID, 16), lambda i: (0, 0)),
            pl.BlockSpec((8, 16), lambda i: (0, 0)),
        ],
        out_specs=[
            pl.BlockSpec((RB, 80), lambda i: (i, 0)),
            pl.BlockSpec((RB, 16), lambda i: (i, 0)),
        ],
        out_shape=[
            jax.ShapeDtypeStruct((N_PAD, 80), jnp.bfloat16),
            jax.ShapeDtypeStruct((N_PAD, 16), jnp.float32),
        ],
    )


def _mid_call():
    def body(num_ref, den_ref, b_ref, e_ref, p_ref, w_ref, a_ref, bb_ref,
             ei_ref, fb_ref, er_ref):
        nm = num_ref[0] + num_ref[1]
        dn = den_ref[0] + den_ref[1]
        nmp = jnp.dot(nm, p_ref[...], preferred_element_type=jnp.float32)
        den64 = jnp.dot(dn, e_ref[...], preferred_element_type=jnp.float32)
        v = nmp / (den64 + 1e-9) + b_ref[...]
        h = jnp.where(v > 0, v, jnp.exp(v) - 1.0)
        feat = jnp.dot(h, w_ref[...], preferred_element_type=jnp.float32)
        el = jnp.dot(feat, a_ref[...], preferred_element_type=jnp.float32)
        er = jnp.dot(feat, bb_ref[...], preferred_element_type=jnp.float32)
        eli = jnp.dot(el, ei_ref[...], preferred_element_type=jnp.float32)
        fb_ref[...] = jnp.concatenate([feat, eli], axis=1).astype(jnp.bfloat16)
        er_ref[...] = er

    return pl.pallas_call(
        body,
        grid=(N_PAD // RB,),
        in_specs=[
            pl.BlockSpec((2, RB, HID), lambda i: (0, i, 0)),
            pl.BlockSpec((2, RB, 16), lambda i: (0, i, 0)),
            pl.BlockSpec((1, HID), lambda i: (0, 0)),
            pl.BlockSpec((16, HID), lambda i: (0, 0)),
            pl.BlockSpec((HID, HID), lambda i: (0, 0)),
            pl.BlockSpec((HID, HID), lambda i: (0, 0)),
            pl.BlockSpec((HID, 8), lambda i: (0, 0)),
            pl.BlockSpec((HID, 16), lambda i: (0, 0)),
            pl.BlockSpec((8, 16), lambda i: (0, 0)),
        ],
        out_specs=[
            pl.BlockSpec((RB, 80), lambda i: (i, 0)),
            pl.BlockSpec((RB, 16), lambda i: (i, 0)),
        ],
        out_shape=[
            jax.ShapeDtypeStruct((N_PAD, 80), jnp.bfloat16),
            jax.ShapeDtypeStruct((N_PAD, 16), jnp.float32),
        ],
    )


def _norm_call(apply_elu):
    def body(num_ref, den_ref, b_ref, e_ref, p_ref, o_ref):
        nm = num_ref[0] + num_ref[1]
        dn = den_ref[0] + den_ref[1]
        nmp = jnp.dot(nm, p_ref[...], preferred_element_type=jnp.float32)
        den64 = jnp.dot(dn, e_ref[...], preferred_element_type=jnp.float32)
        v = nmp / (den64 + 1e-9) + b_ref[...]
        if apply_elu:
            v = jnp.where(v > 0, v, jnp.exp(v) - 1.0)
        o_ref[...] = v

    return pl.pallas_call(
        body,
        grid=(N_PAD // RB,),
        in_specs=[
            pl.BlockSpec((2, RB, HID), lambda i: (0, i, 0)),
            pl.BlockSpec((2, RB, 16), lambda i: (0, i, 0)),
            pl.BlockSpec((1, HID), lambda i: (0, 0)),
            pl.BlockSpec((16, HID), lambda i: (0, 0)),
            pl.BlockSpec((HID, HID), lambda i: (0, 0)),
        ],
        out_specs=pl.BlockSpec((RB, HID), lambda i: (i, 0)),
        out_shape=jax.ShapeDtypeStruct((N_PAD, HID), jnp.float32),
    )


_EXP_A = False
_EXP_B = False
def _edge_call(mode):
    mesh = plsc.VectorSubcoreMesh(core_axis_name="c", subcore_axis_name="s")

    @functools.partial(
        pl.kernel,
        out_type=(
            jax.ShapeDtypeStruct((NC, N_PAD, 64), jnp.float32),
            jax.ShapeDtypeStruct((NC, N_PAD, 16), jnp.float32),
        ),
        mesh=mesh,
        compiler_params=pltpu.CompilerParams(
            needs_layout_passes=False, use_tc_tiling_on_sc=False),
        scratch_types=[
            pltpu.VMEM((NSUB, CH), jnp.int32),
            pltpu.VMEM((NSUB, CH), jnp.int32),
            pltpu.VMEM((CE, 80), jnp.bfloat16),
            pltpu.VMEM((CE, 16), jnp.float32),
            pltpu.VMEM((CE, 16), jnp.float32),
            pltpu.VMEM((CE, 64), jnp.float32),
            pltpu.VMEM((NSUB, CH), jnp.int32),
            pltpu.VMEM((NSUB, CH), jnp.int32),
            pltpu.VMEM((CE, 80), jnp.bfloat16),
            pltpu.VMEM((CE, 16), jnp.float32),
            pltpu.VMEM((CE, 16), jnp.float32),
            pltpu.VMEM((CE, 64), jnp.float32),
            pltpu.VMEM_SHARED((N_PAD, 64), jnp.float32),
            pltpu.VMEM_SHARED((N_PAD, 16), jnp.float32),
            pltpu.SemaphoreType.DMA,
            pltpu.SemaphoreType.DMA,
            pltpu.SemaphoreType.DMA,
            pltpu.SemaphoreType.DMA,
        ],
    )
    def k(src_h, dst_h, fe_h, er_h, num_o, den_o,
          src_v0, dst_v0, fe_v0, er_v0, w_v0, msg_v0,
          src_v1, dst_v1, fe_v1, er_v1, w_v1, msg_v1,
          num_sp, den_sp, gsem0, gsem1, ssem0, ssem1):
        c = lax.axis_index("c")
        s = lax.axis_index("s")
        wid = s * NC + c
        ebase = wid * EPW
        zero16 = jnp.zeros((16,), jnp.float32)
        iota = lax.iota(jnp.int32, 16)
        if mode == 1:
            patt = [8 + 4 * q + (iota >> 2) for q in range(2)]
        else:
            patt = [8 + jnp.zeros((16,), jnp.int32) for _ in range(2)]
        bufs = [(src_v0, dst_v0, fe_v0, er_v0, w_v0, msg_v0, gsem0, ssem0),
                (src_v1, dst_v1, fe_v1, er_v1, w_v1, msg_v1, gsem1, ssem1)]

        @plsc.parallel_loop(0, CE * 4, unroll=8)
        def znloop(i):
            msg_v0[i >> 2, pl.ds((i & 3) * 16, 16)] = zero16

        @plsc.parallel_loop(0, CE, unroll=8)
        def zdloop(i):
            w_v0[i, :] = zero16

        row0 = s * ROWS_PT
        done = 0
        while done < ROWS_PT:
            step = min(CE, ROWS_PT - done)
            pltpu.sync_copy(msg_v0.at[pl.ds(0, step)],
                            num_sp.at[pl.ds(row0 + done, step)])
            pltpu.sync_copy(w_v0.at[pl.ds(0, step)],
                            den_sp.at[pl.ds(row0 + done, step)])
            done += step
        plsc.subcore_barrier()

        def load_idx(g, b):
            src_v, dst_v = bufs[b][0], bufs[b][1]
            for j in range(NSUB):
                base = ebase + g * CE + j * CH
                pltpu.sync_copy(src_h.at[pl.ds(base, CH)], src_v.at[j])
                pltpu.sync_copy(dst_h.at[pl.ds(base, CH)], dst_v.at[j])

        def gathers(b):
            src_v, dst_v, fe_v, er_v = bufs[b][0], bufs[b][1], bufs[b][2], bufs[b][3]
            gsem = bufs[b][6]
            out = []
            for j in range(NSUB):
                out.append(pltpu.make_async_copy(
                    fe_h.at[src_v.at[j]], fe_v.at[pl.ds(j * CH, CH)], gsem))
                out.append(pltpu.make_async_copy(
                    er_h.at[dst_v.at[j]], er_v.at[pl.ds(j * CH, CH)], gsem))
            return out

        def scatters(b):
            dst_v, w_v, msg_v = bufs[b][1], bufs[b][4], bufs[b][5]
            ssem = bufs[b][7]
            out = []
            for j in range(NSUB):
                out.append(pltpu.make_async_copy(
                    msg_v.at[pl.ds(j * CH, CH)], num_sp.at[dst_v.at[j]], ssem))
                out.append(pltpu.make_async_copy(
                    w_v.at[pl.ds(j * CH, CH)], den_sp.at[dst_v.at[j]], ssem))
            return out

        def compute(b):
            fe_v, er_v, w_v, msg_v = bufs[b][2], bufs[b][3], bufs[b][4], bufs[b][5]

            @plsc.parallel_loop(0, CE, unroll=4)
            def rows(r):
                ue = fe_v[r, pl.ds(48, 32)]
                ea, _eb = plsc.unpack(ue, format=plsc.PackFormat.INTERLEAVED,
                                      preferred_element_type=jnp.float32)
                z = ea + er_v[r, :]
                w = jnp.exp(jnp.where(z > 0, z, 0.2 * z))
                w_v[r, :] = w
                for q in range(2):
                    uf = fe_v[r, pl.ds(q * 32, 32)]
                    fa, fb2 = plsc.unpack(uf, format=plsc.PackFormat.INTERLEAVED,
                                          preferred_element_type=jnp.float32)
                    wq = jnp.take_along_axis(w, patt[q], axis=0)
                    msg_v[r, pl.ds(q * 32, 16)] = fa * wq
                    msg_v[r, pl.ds(q * 32 + 16, 16)] = fb2 * wq

        load_idx(0, 0)
        for cp in gathers(0):
            cp.start()

        def halfstep(i, b):
            g = 2 * i + b
            nb2 = 1 - b

            @pl.when(g >= 1)
            def _():
                for cp in scatters(nb2):
                    cp.wait()

            @pl.when(g + 1 < NCHUNK)
            def _():
                load_idx(g + 1, nb2)
                for cp in gathers(nb2):
                    cp.start()

            for cp in gathers(b):
                cp.wait()
            compute(b)
            for cp in scatters(b):
                cp.start(add=True)

        def pipe(i, _):
            halfstep(i, 0)
            halfstep(i, 1)
            return 0
        lax.fori_loop(0, NCHUNK // 2, pipe, 0)

        for cp in scatters((NCHUNK - 1) & 1):
            cp.wait()
        plsc.subcore_barrier()

        pltpu.sync_copy(num_sp.at[pl.ds(row0, ROWS_PT)],
                        num_o.at[c, pl.ds(row0, ROWS_PT)])
        pltpu.sync_copy(den_sp.at[pl.ds(row0, ROWS_PT)],
                        den_o.at[c, pl.ds(row0, ROWS_PT)])

    return k


_E16_L1 = np.zeros((16, HID), np.float32)
for _h in range(8):
    _E16_L1[8 + _h, _h * 8:(_h + 1) * 8] = 1.0
_E16_L2 = np.zeros((16, HID), np.float32)
_E16_L2[8, :] = 1.0

_PERM = np.zeros((HID, HID), np.float32)
for _c in range(HID):
    _q, _rc = _c // 32, _c % 32
    _f = 32 * _q + (2 * _rc if _rc < 16 else 2 * (_rc - 16) + 1)
    _PERM[_c, _f] = 1.0

_EI = np.zeros((8, 16), np.float32)
for _j in range(8):
    _EI[_j, 2 * _j] = 1.0
    _EI[_j, 2 * _j + 1] = 1.0


def _blockdiag(a):
    return (jnp.eye(8, dtype=a.dtype)[:, None, :] * a[:, :, None]).reshape(HID, 8)


def kernel(x, edge_index, W1, al1, ar1, b1, W2, al2, ar2, b2):
    src = jnp.concatenate(
        [edge_index[0].astype(jnp.int32), jnp.full((E_PAD - E,), N, jnp.int32)])
    dst = jnp.concatenate(
        [edge_index[1].astype(jnp.int32), jnp.full((E_PAD - E,), N, jnp.int32)])
    xp = jnp.zeros((N_PAD, D), jnp.float32).at[:N].set(x)

    A1 = _blockdiag(al1)
    B1 = jnp.concatenate([_blockdiag(ar1)] * 2, axis=1)
    A2 = jnp.tile(al2.T, (1, 8))
    B2 = jnp.tile(ar2.T, (1, 16))
    E1 = jnp.asarray(_E16_L1)
    E2 = jnp.asarray(_E16_L2)
    P = jnp.asarray(_PERM)
    EI = jnp.asarray(_EI)

    fb1, er1 = _proj_call(D)(xp, W1, A1, B1, EI)
    num1, den1 = _edge_call(1)(src, dst, fb1, er1)
    fb2, er2 = _mid_call()(num1, den1, b1.reshape(1, HID), E1, P, W2, A2, B2, EI)
    num2, den2 = _edge_call(2)(src, dst, fb2, er2)
    out = _norm_call(False)(num2, den2, b2.reshape(1, HID), E2, P)
    return out[:N]
